# baseline (speedup 1.0000x reference)
import jax
import jax.numpy as jnp
import numpy as np
from jax.experimental import pallas as pl
from jax.experimental.pallas import tpu as pltpu

_B, _A, _NOBJ = 32, 16320, 50
_R, _C = 64, 255
_NCH, _CS = 4, 16
_IPS = 16
_TH = 0.5
_V0, _V1 = 0.1, 0.2
_NEG_POS_RATIO = 3

_BITS_2 = int(np.float32(2.0).view(np.int32))
_BITS_TH = int(np.float32(_TH).view(np.int32))
_MASK_HI = ~np.int32(63)


def _match_image(tgt_ref, loc_ref, conf_ref, anc_ref, i,
                 ax1, ay1, ax2, ay2, area_a):
    keys = [jnp.full((_CS, _C), -1, dtype=jnp.int32) for _ in range(_NCH)]
    for t in range(_NOBJ):
        tx1 = tgt_ref[i, t, 0]
        ty1 = tgt_ref[i, t, 1]
        tx2 = tgt_ref[i, t, 2]
        ty2 = tgt_ref[i, t, 3]
        area_t = (tx2 - tx1) * (ty2 - ty1)
        kts = []
        pmax = None
        for k in range(_NCH):
            iw = jnp.maximum(jnp.minimum(tx2, ax2[k]) - jnp.maximum(tx1, ax1[k]), 0.0)
            ih = jnp.maximum(jnp.minimum(ty2, ay2[k]) - jnp.maximum(ty1, ay1[k]), 0.0)
            inter = iw * ih
            iou = inter / (area_t + area_a[k] - inter)
            kt = (jax.lax.bitcast_convert_type(iou, jnp.int32) & _MASK_HI) | (63 - t)
            kts.append(kt)
            pmax = kt if pmax is None else jnp.maximum(pmax, kt)
        rm = jnp.max(pmax, keepdims=True)
        rm = jnp.where(rm > 63, rm, -2)
        ovkey = jnp.int32(_BITS_2 | t)
        for k in range(_NCH):
            keys[k] = jnp.maximum(
                keys[k], jnp.where(kts[k] == rm, ovkey, kts[k]))

    num_pos = jnp.float32(0.0)
    pos = []
    tsel = []
    for k in range(_NCH):
        kk = keys[k]
        p = (kk & _MASK_HI) >= _BITS_TH
        pos.append(p)
        low = kk & 63
        tsel.append(jnp.where(kk >= _BITS_2, low, 63 - low))
        num_pos += jnp.sum(p.astype(jnp.float32))

    lsum = jnp.float32(0.0)
    for k in range(_NCH):
        sx = jnp.zeros((_CS, _C), jnp.float32)
        dx = jnp.ones((_CS, _C), jnp.float32)
        sy = jnp.zeros((_CS, _C), jnp.float32)
        dy = jnp.ones((_CS, _C), jnp.float32)
        for t in range(_NOBJ):
            upd = tsel[k] == t
            sx = jnp.where(upd, tgt_ref[i, t, 0] + tgt_ref[i, t, 2], sx)
            dx = jnp.where(upd, tgt_ref[i, t, 2] - tgt_ref[i, t, 0], dx)
            sy = jnp.where(upd, tgt_ref[i, t, 1] + tgt_ref[i, t, 3], sy)
            dy = jnp.where(upd, tgt_ref[i, t, 3] - tgt_ref[i, t, 1], dy)
        cx = anc_ref[0, 16 * k:16 * (k + 1), :]
        cy = anc_ref[1, 16 * k:16 * (k + 1), :]
        aw = anc_ref[2, 16 * k:16 * (k + 1), :]
        ah = anc_ref[3, 16 * k:16 * (k + 1), :]
        g0 = (sx * 0.5 - cx) / (_V0 * aw)
        g1 = (sy * 0.5 - cy) / (_V0 * ah)
        g2 = jnp.log(dx / aw) / _V1
        g3 = jnp.log(dy / ah) / _V1
        sl1_tot = jnp.zeros((_CS, _C), jnp.float32)
        for j, g in enumerate((g0, g1, g2, g3)):
            d = loc_ref[i, j, 16 * k:16 * (k + 1), :] - g
            ad = jnp.abs(d)
            sl1_tot += jnp.where(ad < 1.0, 0.5 * d * d, ad - 0.5)
        lsum += jnp.sum(jnp.where(pos[k], sl1_tot, 0.0))

    ce_pos = jnp.float32(0.0)
    vs = []
    us = []
    for k in range(_NCH):
        c0 = conf_ref[i, 0, 16 * k:16 * (k + 1), :]
        c1 = conf_ref[i, 1, 16 * k:16 * (k + 1), :]
        cm = jnp.maximum(c0, c1)
        lse = cm + jnp.log(jnp.exp(c0 - cm) + jnp.exp(c1 - cm))
        ce_pos += jnp.sum(jnp.where(pos[k], lse - c1, 0.0))
        v = jnp.where(pos[k], 0.0, lse - c0)
        vs.append(v)
        us.append(jax.lax.bitcast_convert_type(v, jnp.int32))
    return num_pos, lsum, ce_pos, vs, us


def _arm_body(tgt_ref, loc_ref, conf_ref, anc_ref, out_l_ref,
              out_c_ref, acc):
    b = pl.program_id(0)

    @pl.when(b == 0)
    def _init():
        acc[0] = 0.0
        acc[1] = 0.0
        acc[2] = 0.0

    def chunk(ref, *lead):
        return [ref[lead + (slice(16 * k, 16 * (k + 1)), slice(None))]
                for k in range(_NCH)]

    ax1 = chunk(anc_ref, 4)
    ay1 = chunk(anc_ref, 5)
    ax2 = chunk(anc_ref, 6)
    ay2 = chunk(anc_ref, 7)
    area_a = chunk(anc_ref, 8)

    per_img = [_match_image(tgt_ref, loc_ref, conf_ref, anc_ref, i,
                            ax1, ay1, ax2, ay2, area_a)
               for i in range(_IPS)]

    knegs = []
    for num_pos, _, _, _, _ in per_img:
        np_i = num_pos.astype(jnp.int32)
        knegs.append(jnp.minimum(_NEG_POS_RATIO * np_i, _A - np_i))

    def bisect(_, lohis):
        out = []
        for (lo, hi), kneg, (_, _, _, _, us) in zip(lohis, knegs, per_img):
            mid = lo + (hi - lo) // 2
            cs = (us[0] > mid).astype(jnp.int32)
            for k in range(1, _NCH):
                cs += (us[k] > mid).astype(jnp.int32)
            sat = jnp.sum(cs) < kneg
            out.append((jnp.where(sat, lo, mid), jnp.where(sat, mid, hi)))
        return tuple(out)

    init = tuple((jnp.int32(-1), jnp.int32(2**31 - 2)) for _ in range(_IPS))
    lohis = jax.lax.fori_loop(0, 31, bisect, init)

    d_l = jnp.float32(0.0)
    d_c = jnp.float32(0.0)
    d_np = jnp.float32(0.0)
    for (_, thr), kneg, (num_pos, lsum, ce_pos, vs, us) in zip(
            lohis, knegs, per_img):
        sum_gt = jnp.float32(0.0)
        cnt_gt = jnp.int32(0)
        for k in range(_NCH):
            gt = us[k] > thr
            cnt_gt += jnp.sum(gt.astype(jnp.int32))
            sum_gt += jnp.sum(jnp.where(gt, vs[k], 0.0))
        thr_f = jax.lax.bitcast_convert_type(thr, jnp.float32)
        tie = (kneg - cnt_gt).astype(jnp.float32) * thr_f
        topk = jnp.where(kneg > 0, sum_gt + tie, 0.0)
        d_l += lsum
        d_c += ce_pos + topk
        d_np += num_pos

    acc[0] = acc[0] + d_l
    acc[1] = acc[1] + d_c
    acc[2] = acc[2] + d_np

    @pl.when(b == _B // _IPS - 1)
    def _fin():
        out_l_ref[...] = jnp.full((1, 1), acc[0] / acc[2], dtype=jnp.float32)
        out_c_ref[...] = jnp.full((1, 1), acc[1] / acc[2], dtype=jnp.float32)


def kernel(loc_pred, conf_pred, anchors, targets):
    loc_t = loc_pred.transpose(0, 2, 1).reshape(_B, 4, _R, _C)
    conf_t = conf_pred.transpose(0, 2, 1).reshape(_B, 2, _R, _C)
    cx, cy, aw, ah = anchors[:, 0], anchors[:, 1], anchors[:, 2], anchors[:, 3]
    anc_pack = jnp.stack([
        cx, cy, aw, ah,
        cx - aw * 0.5, cy - ah * 0.5, cx + aw * 0.5, cy + ah * 0.5,
        aw * ah,
    ]).reshape(9, _R, _C)

    out = pl.pallas_call(
        _arm_body,
        grid=(_B // _IPS,),
        in_specs=[
            pl.BlockSpec((_IPS, _NOBJ, 5), lambda b: (b, 0, 0),
                         memory_space=pltpu.SMEM),
            pl.BlockSpec((_IPS, 4, _R, _C), lambda b: (b, 0, 0, 0)),
            pl.BlockSpec((_IPS, 2, _R, _C), lambda b: (b, 0, 0, 0)),
            pl.BlockSpec((9, _R, _C), lambda b: (0, 0, 0)),
        ],
        out_specs=[
            pl.BlockSpec((1, 1), lambda b: (0, 0)),
            pl.BlockSpec((1, 1), lambda b: (0, 0)),
        ],
        out_shape=[
            jax.ShapeDtypeStruct((1, 1), jnp.float32),
            jax.ShapeDtypeStruct((1, 1), jnp.float32),
        ],
        scratch_shapes=[pltpu.SMEM((3,), jnp.float32)],
    )(targets, loc_t, conf_t, anc_pack)
    return out[0].reshape(()), out[1].reshape(())


# speedup vs baseline: 110.0583x; 110.0583x over previous
import jax
import jax.numpy as jnp
import numpy as np
from jax.experimental import pallas as pl
from jax.experimental.pallas import tpu as pltpu

_B, _A, _NOBJ = 32, 16320, 50
_R, _C = 64, 255
_NCH, _CS = 4, 16
_IPS = 8
_TH = 0.5
_V0, _V1 = 0.1, 0.2
_NEG_POS_RATIO = 3

_BITS_2 = int(np.float32(2.0).view(np.int32))
_BITS_TH = int(np.float32(_TH).view(np.int32))
_MASK_HI = ~np.int32(63)


def _match_image(tgt_ref, loc_ref, conf_ref, anc_ref, i,
                 ax1, ay1, ax2, ay2, area_a):
    keys = [jnp.full((_CS, _C), -1, dtype=jnp.int32) for _ in range(_NCH)]
    for t in range(_NOBJ):
        tx1 = tgt_ref[i, t, 0]
        ty1 = tgt_ref[i, t, 1]
        tx2 = tgt_ref[i, t, 2]
        ty2 = tgt_ref[i, t, 3]
        area_t = (tx2 - tx1) * (ty2 - ty1)
        kts = []
        pmax = None
        for k in range(_NCH):
            iw = jnp.maximum(jnp.minimum(tx2, ax2[k]) - jnp.maximum(tx1, ax1[k]), 0.0)
            ih = jnp.maximum(jnp.minimum(ty2, ay2[k]) - jnp.maximum(ty1, ay1[k]), 0.0)
            inter = iw * ih
            iou = inter / (area_t + area_a[k] - inter)
            kt = (jax.lax.bitcast_convert_type(iou, jnp.int32) & _MASK_HI) | (63 - t)
            kts.append(kt)
            pmax = kt if pmax is None else jnp.maximum(pmax, kt)
        rm = jnp.max(pmax, keepdims=True)
        rm = jnp.where(rm > 63, rm, -2)
        ovkey = jnp.int32(_BITS_2 | t)
        for k in range(_NCH):
            keys[k] = jnp.maximum(
                keys[k], jnp.where(kts[k] == rm, ovkey, kts[k]))

    num_pos = jnp.float32(0.0)
    pos = []
    tsel = []
    for k in range(_NCH):
        kk = keys[k]
        p = (kk & _MASK_HI) >= _BITS_TH
        pos.append(p)
        low = kk & 63
        tsel.append(jnp.where(kk >= _BITS_2, low, 63 - low))
        num_pos += jnp.sum(p.astype(jnp.float32))

    lsum = jnp.float32(0.0)
    for k in range(_NCH):
        sx = jnp.zeros((_CS, _C), jnp.float32)
        dx = jnp.ones((_CS, _C), jnp.float32)
        sy = jnp.zeros((_CS, _C), jnp.float32)
        dy = jnp.ones((_CS, _C), jnp.float32)
        for t in range(_NOBJ):
            upd = tsel[k] == t
            sx = jnp.where(upd, tgt_ref[i, t, 0] + tgt_ref[i, t, 2], sx)
            dx = jnp.where(upd, tgt_ref[i, t, 2] - tgt_ref[i, t, 0], dx)
            sy = jnp.where(upd, tgt_ref[i, t, 1] + tgt_ref[i, t, 3], sy)
            dy = jnp.where(upd, tgt_ref[i, t, 3] - tgt_ref[i, t, 1], dy)
        cx = anc_ref[0, 16 * k:16 * (k + 1), :]
        cy = anc_ref[1, 16 * k:16 * (k + 1), :]
        aw = anc_ref[2, 16 * k:16 * (k + 1), :]
        ah = anc_ref[3, 16 * k:16 * (k + 1), :]
        g0 = (sx * 0.5 - cx) / (_V0 * aw)
        g1 = (sy * 0.5 - cy) / (_V0 * ah)
        g2 = jnp.log(dx / aw) / _V1
        g3 = jnp.log(dy / ah) / _V1
        sl1_tot = jnp.zeros((_CS, _C), jnp.float32)
        for j, g in enumerate((g0, g1, g2, g3)):
            d = loc_ref[i, j, 16 * k:16 * (k + 1), :] - g
            ad = jnp.abs(d)
            sl1_tot += jnp.where(ad < 1.0, 0.5 * d * d, ad - 0.5)
        lsum += jnp.sum(jnp.where(pos[k], sl1_tot, 0.0))

    ce_pos = jnp.float32(0.0)
    vs = []
    us = []
    for k in range(_NCH):
        c0 = conf_ref[i, 0, 16 * k:16 * (k + 1), :]
        c1 = conf_ref[i, 1, 16 * k:16 * (k + 1), :]
        cm = jnp.maximum(c0, c1)
        lse = cm + jnp.log(jnp.exp(c0 - cm) + jnp.exp(c1 - cm))
        ce_pos += jnp.sum(jnp.where(pos[k], lse - c1, 0.0))
        v = jnp.where(pos[k], 0.0, lse - c0)
        vs.append(v)
        us.append(jax.lax.bitcast_convert_type(v, jnp.int32))
    return num_pos, lsum, ce_pos, vs, us


def _arm_body(tgt_ref, loc_ref, conf_ref, anc_ref, out_l_ref,
              out_c_ref, acc):
    b = pl.program_id(0)

    @pl.when(b == 0)
    def _init():
        acc[0] = 0.0
        acc[1] = 0.0
        acc[2] = 0.0

    def chunk(ref, *lead):
        return [ref[lead + (slice(16 * k, 16 * (k + 1)), slice(None))]
                for k in range(_NCH)]

    ax1 = chunk(anc_ref, 4)
    ay1 = chunk(anc_ref, 5)
    ax2 = chunk(anc_ref, 6)
    ay2 = chunk(anc_ref, 7)
    area_a = chunk(anc_ref, 8)

    per_img = [_match_image(tgt_ref, loc_ref, conf_ref, anc_ref, i,
                            ax1, ay1, ax2, ay2, area_a)
               for i in range(_IPS)]

    knegs = []
    for num_pos, _, _, _, _ in per_img:
        np_i = num_pos.astype(jnp.int32)
        knegs.append(jnp.minimum(_NEG_POS_RATIO * np_i, _A - np_i))

    def bisect(_, lohis):
        out = []
        for (lo, hi), kneg, (_, _, _, _, us) in zip(lohis, knegs, per_img):
            mid = lo + (hi - lo) // 2
            cs = (us[0] > mid).astype(jnp.int32)
            for k in range(1, _NCH):
                cs += (us[k] > mid).astype(jnp.int32)
            sat = jnp.sum(cs) < kneg
            out.append((jnp.where(sat, lo, mid), jnp.where(sat, mid, hi)))
        return tuple(out)

    init = tuple((jnp.int32(-1), jnp.int32(2**31 - 2)) for _ in range(_IPS))
    lohis = jax.lax.fori_loop(0, 31, bisect, init)

    d_l = jnp.float32(0.0)
    d_c = jnp.float32(0.0)
    d_np = jnp.float32(0.0)
    for (_, thr), kneg, (num_pos, lsum, ce_pos, vs, us) in zip(
            lohis, knegs, per_img):
        sum_gt = jnp.float32(0.0)
        cnt_gt = jnp.int32(0)
        for k in range(_NCH):
            gt = us[k] > thr
            cnt_gt += jnp.sum(gt.astype(jnp.int32))
            sum_gt += jnp.sum(jnp.where(gt, vs[k], 0.0))
        thr_f = jax.lax.bitcast_convert_type(thr, jnp.float32)
        tie = (kneg - cnt_gt).astype(jnp.float32) * thr_f
        topk = jnp.where(kneg > 0, sum_gt + tie, 0.0)
        d_l += lsum
        d_c += ce_pos + topk
        d_np += num_pos

    acc[0] = acc[0] + d_l
    acc[1] = acc[1] + d_c
    acc[2] = acc[2] + d_np

    @pl.when(b == _B // _IPS - 1)
    def _fin():
        out_l_ref[...] = jnp.full((1, 1), acc[0] / acc[2], dtype=jnp.float32)
        out_c_ref[...] = jnp.full((1, 1), acc[1] / acc[2], dtype=jnp.float32)


def kernel(loc_pred, conf_pred, anchors, targets):
    loc_t = loc_pred.transpose(0, 2, 1).reshape(_B, 4, _R, _C)
    conf_t = conf_pred.transpose(0, 2, 1).reshape(_B, 2, _R, _C)
    cx, cy, aw, ah = anchors[:, 0], anchors[:, 1], anchors[:, 2], anchors[:, 3]
    anc_pack = jnp.stack([
        cx, cy, aw, ah,
        cx - aw * 0.5, cy - ah * 0.5, cx + aw * 0.5, cy + ah * 0.5,
        aw * ah,
    ]).reshape(9, _R, _C)

    out = pl.pallas_call(
        _arm_body,
        grid=(_B // _IPS,),
        in_specs=[
            pl.BlockSpec((_IPS, _NOBJ, 5), lambda b: (b, 0, 0),
                         memory_space=pltpu.SMEM),
            pl.BlockSpec((_IPS, 4, _R, _C), lambda b: (b, 0, 0, 0)),
            pl.BlockSpec((_IPS, 2, _R, _C), lambda b: (b, 0, 0, 0)),
            pl.BlockSpec((9, _R, _C), lambda b: (0, 0, 0)),
        ],
        out_specs=[
            pl.BlockSpec((1, 1), lambda b: (0, 0)),
            pl.BlockSpec((1, 1), lambda b: (0, 0)),
        ],
        out_shape=[
            jax.ShapeDtypeStruct((1, 1), jnp.float32),
            jax.ShapeDtypeStruct((1, 1), jnp.float32),
        ],
        scratch_shapes=[pltpu.SMEM((3,), jnp.float32)],
    )(targets, loc_t, conf_t, anc_pack)
    return out[0].reshape(()), out[1].reshape(())
